# baseline (speedup 1.0000x reference)
import functools

import jax
import jax.numpy as jnp
from jax import lax
from jax.experimental import pallas as pl
from jax.experimental.pallas import tpu as pltpu
from jax.experimental.pallas import tpu_sc as plsc

NA = 10000
NCO = 5000
E = 320000
B = 64
NMAX = 450
K1 = 75
D = 256
DIMS = [96, 96, 128, 256]

NROW = 512
TRASH = B * NROW
DENSE_ROWS = B * NROW + 512
NA_PAD = 10240
NCO_PAD = 5120

SC_CORES = 2
SC_SUBCORES = 16
SC_WORKERS = SC_CORES * SC_SUBCORES
EDGE_CHUNK = 128
E_PAD = 327680


def _sc_mesh():
  return plsc.VectorSubcoreMesh(core_axis_name="c", subcore_axis_name="s")


def _sc_degrees(dst_c2a, dst_a2c, ones_chunk, zeros_a, zeros_c):
  nchunk = E // EDGE_CHUNK
  per_w = -(-nchunk // SC_WORKERS)
  rows_a = NA_PAD // SC_SUBCORES
  rows_c = NCO_PAD // SC_SUBCORES

  @functools.partial(
      pl.kernel,
      out_type=jax.ShapeDtypeStruct((SC_CORES, NA_PAD + NCO_PAD, 16),
                                    jnp.float32),
      mesh=_sc_mesh(),
      scratch_types=[
          pltpu.VMEM((EDGE_CHUNK,), jnp.int32),
          pltpu.VMEM((EDGE_CHUNK, 16), jnp.float32),
          pltpu.VMEM_SHARED((NA_PAD, 16), jnp.float32),
          pltpu.VMEM_SHARED((NCO_PAD, 16), jnp.float32),
      ],
  )
  def k(dca_h, dac_h, ones_h, za_h, zc_h, out_h,
        idx_v, ones_v, acc_a, acc_c):
    c = lax.axis_index("c")
    s = lax.axis_index("s")
    wid = c * SC_SUBCORES + s
    pltpu.sync_copy(ones_h, ones_v)
    pltpu.sync_copy(za_h.at[pl.ds(s * rows_a, rows_a)],
                    acc_a.at[pl.ds(s * rows_a, rows_a)])
    pltpu.sync_copy(zc_h.at[pl.ds(s * rows_c, rows_c)],
                    acc_c.at[pl.ds(s * rows_c, rows_c)])
    plsc.subcore_barrier()

    def body(g, carry):
      ch = wid + SC_WORKERS * g

      @pl.when(ch < nchunk)
      def _():
        off = ch * EDGE_CHUNK
        pltpu.sync_copy(dca_h.at[pl.ds(off, EDGE_CHUNK)], idx_v)
        pltpu.sync_copy(ones_v, acc_a.at[idx_v], add=True)
        pltpu.sync_copy(dac_h.at[pl.ds(off, EDGE_CHUNK)], idx_v)
        pltpu.sync_copy(ones_v, acc_c.at[idx_v], add=True)

      return carry

    lax.fori_loop(0, per_w, body, 0)
    plsc.subcore_barrier()
    pltpu.sync_copy(acc_a.at[pl.ds(s * rows_a, rows_a)],
                    out_h.at[c, pl.ds(s * rows_a, rows_a)])
    pltpu.sync_copy(acc_c.at[pl.ds(s * rows_c, rows_c)],
                    out_h.at[c, pl.ds(NA_PAD + s * rows_c, rows_c)])

  return k(dst_c2a, dst_a2c, ones_chunk, zeros_a, zeros_c)


def _sc_segsum(table, s2d, d2d, n_pad, grp, idxch):
  NCH = E_PAD // EDGE_CHUNK // SC_WORKERS
  rows_sub = n_pad // SC_SUBCORES

  @functools.partial(
      pl.kernel,
      out_type=jax.ShapeDtypeStruct((SC_CORES, n_pad, 128), jnp.float32),
      mesh=_sc_mesh(),
      scratch_types=[
          pltpu.VMEM((idxch, EDGE_CHUNK), jnp.int32),
          pltpu.VMEM((idxch, EDGE_CHUNK), jnp.int32),
          pltpu.VMEM((grp, EDGE_CHUNK, 128), jnp.float32),
          pltpu.VMEM_SHARED((n_pad, 128), jnp.float32),
      ] + [pltpu.SemaphoreType.DMA] * grp,
  )
  def k(table_h, src_h, dst_h, out_h, src_v, dst_v, rows_v, acc, *sems):
    c = lax.axis_index("c")
    s = lax.axis_index("s")
    wid = c * SC_SUBCORES + s
    cbase = wid * NCH

    z16 = jnp.zeros((16,), jnp.float32)

    def zrow(r, carry):
      for cc in range(8):
        rows_v[0, r, pl.ds(cc * 16, 16)] = z16
      return carry

    lax.fori_loop(0, EDGE_CHUNK, zrow, 0)
    for j in range(rows_sub // EDGE_CHUNK):
      pltpu.sync_copy(
          rows_v.at[0],
          acc.at[pl.ds(s * rows_sub + j * EDGE_CHUNK, EDGE_CHUNK)])
    plsc.subcore_barrier()

    for h in range(NCH // idxch):
      hb = cbase + h * idxch
      pltpu.sync_copy(src_h.at[pl.ds(hb, idxch)], src_v)
      pltpu.sync_copy(dst_h.at[pl.ds(hb, idxch)], dst_v)

      def group(g, carry):
        ch0 = g * grp
        cps = []
        for t in range(grp):
          cps.append(pltpu.async_copy(table_h.at[src_v.at[ch0 + t]],
                                      rows_v.at[t], sems[t]))
        for t in range(grp):
          cps[t].wait()
          pltpu.sync_copy(rows_v.at[t], acc.at[dst_v.at[ch0 + t]], add=True)
        return carry

      lax.fori_loop(0, idxch // grp, group, 0)
    plsc.subcore_barrier()
    pltpu.sync_copy(acc.at[pl.ds(s * rows_sub, rows_sub)],
                    out_h.at[c, pl.ds(s * rows_sub, rows_sub)])

  return k(table, s2d, d2d)


def _sc_pack(xa, xc, idx_a, idx_c):
  CH = 128
  a_full = NA // CH
  a_tail = NA - a_full * CH
  c_full = NCO // CH
  c_tail = NCO - c_full * CH
  a_per_w = -(-a_full // SC_WORKERS)
  c_per_w = -(-c_full // SC_WORKERS)

  @functools.partial(
      pl.kernel,
      out_type=jax.ShapeDtypeStruct((DENSE_ROWS, D), jnp.float32),
      mesh=_sc_mesh(),
      scratch_types=[
          pltpu.VMEM((CH,), jnp.int32),
          pltpu.VMEM((CH, D), jnp.float32),
          pltpu.VMEM((a_tail,), jnp.int32),
          pltpu.VMEM((a_tail, D), jnp.float32),
          pltpu.VMEM((c_tail,), jnp.int32),
          pltpu.VMEM((c_tail, D), jnp.float32),
      ],
  )
  def k(xa_h, xc_h, ia_h, ic_h, out_h,
        idx_v, rows_v, iat_v, rat_v, ict_v, rct_v):
    c = lax.axis_index("c")
    s = lax.axis_index("s")
    wid = c * SC_SUBCORES + s

    def body_a(j, carry):
      ch = wid + SC_WORKERS * j

      @pl.when(ch < a_full)
      def _():
        off = ch * CH
        pltpu.sync_copy(ia_h.at[pl.ds(off, CH)], idx_v)
        pltpu.sync_copy(xa_h.at[pl.ds(off, CH)], rows_v)
        pltpu.sync_copy(rows_v, out_h.at[idx_v])

      return carry

    def body_c(j, carry):
      ch = wid + SC_WORKERS * j

      @pl.when(ch < c_full)
      def _():
        off = ch * CH
        pltpu.sync_copy(ic_h.at[pl.ds(off, CH)], idx_v)
        pltpu.sync_copy(xc_h.at[pl.ds(off, CH)], rows_v)
        pltpu.sync_copy(rows_v, out_h.at[idx_v])

      return carry

    lax.fori_loop(0, a_per_w, body_a, 0)
    lax.fori_loop(0, c_per_w, body_c, 0)

    @pl.when(wid == SC_WORKERS - 1)
    def _():
      pltpu.sync_copy(ia_h.at[pl.ds(a_full * CH, a_tail)], iat_v)
      pltpu.sync_copy(xa_h.at[pl.ds(a_full * CH, a_tail)], rat_v)
      pltpu.sync_copy(rat_v, out_h.at[iat_v])

    @pl.when(wid == SC_WORKERS - 2)
    def _():
      pltpu.sync_copy(ic_h.at[pl.ds(c_full * CH, c_tail)], ict_v)
      pltpu.sync_copy(xc_h.at[pl.ds(c_full * CH, c_tail)], rct_v)
      pltpu.sync_copy(rct_v, out_h.at[ict_v])

  return k(xa, xc, idx_a, idx_c)


def _tc_proj(x, W, b):
  n, di = x.shape
  do = W.shape[1]
  BR = 1000
  grid = n // BR

  def body(x_ref, w_ref, b_ref, o_ref):
    y = jax.lax.dot_general(x_ref[...], w_ref[...], (((1,), (0,)), ((), ())),
                            preferred_element_type=jnp.float32)
    o_ref[...] = jnp.maximum(y + b_ref[...], 0.0)

  return pl.pallas_call(
      body,
      grid=(grid,),
      in_specs=[
          pl.BlockSpec((BR, di), lambda i: (i, 0)),
          pl.BlockSpec((di, do), lambda i: (0, 0)),
          pl.BlockSpec((1, do), lambda i: (0, 0)),
      ],
      out_specs=pl.BlockSpec((BR, do), lambda i: (i, 0)),
      out_shape=jax.ShapeDtypeStruct((n, do), jnp.float32),
  )(x, W, b.reshape(1, do))


def _tc_node_update(x, P, degp, W_self, b_self, W_msg):
  n, di = x.shape
  do = W_self.shape[1]
  BR = 1000
  grid = n // BR

  def body(x_ref, p_ref, d_ref, ws_ref, b_ref, wm_ref, o_ref):
    deg = jnp.maximum(d_ref[0, :, 0:1] + d_ref[1, :, 0:1], 1.0)
    msg = (p_ref[0] + p_ref[1]) / deg
    y = jax.lax.dot_general(x_ref[...], ws_ref[...], (((1,), (0,)), ((), ())),
                            preferred_element_type=jnp.float32)
    y = y + jax.lax.dot_general(msg, wm_ref[...], (((1,), (0,)), ((), ())),
                                preferred_element_type=jnp.float32)
    o_ref[...] = jnp.maximum(y + b_ref[...], 0.0)

  return pl.pallas_call(
      body,
      grid=(grid,),
      in_specs=[
          pl.BlockSpec((BR, di), lambda i: (i, 0)),
          pl.BlockSpec((2, BR, di), lambda i: (0, i, 0)),
          pl.BlockSpec((2, BR, 16), lambda i: (0, i, 0)),
          pl.BlockSpec((di, do), lambda i: (0, 0)),
          pl.BlockSpec((1, do), lambda i: (0, 0)),
          pl.BlockSpec((di, do), lambda i: (0, 0)),
      ],
      out_specs=pl.BlockSpec((BR, do), lambda i: (i, 0)),
      out_shape=jax.ShapeDtypeStruct((n, do), jnp.float32),
  )(x, P, degp, W_self, b_self.reshape(1, do), W_msg)


def _tc_pack_index(batch_atoms, batch_cohps):
  def body(ba_ref, bc_ref, ia_ref, ic_ref, t_ref):
    cols = lax.broadcasted_iota(jnp.int32, (1, B), 1)
    lt = (lax.broadcasted_iota(jnp.int32, (B, B), 0)
          < lax.broadcasted_iota(jnp.int32, (B, B), 1)).astype(jnp.float32)

    ba = ba_ref[...]
    eq_a = (ba == cols).astype(jnp.float32)
    cnt_a = jnp.sum(eq_a, axis=0, keepdims=True)
    starts_a = jax.lax.dot_general(cnt_a, lt, (((1,), (0,)), ((), ())),
                                   preferred_element_type=jnp.float32)
    row_start = jax.lax.dot_general(eq_a, starts_a.reshape(B, 1),
                                    (((1,), (0,)), ((), ())),
                                    preferred_element_type=jnp.float32)
    pos_a = lax.broadcasted_iota(jnp.int32, (NA, 1), 0).astype(jnp.float32) \
        - row_start
    idx_a = jnp.where(pos_a < float(NMAX),
                      ba.astype(jnp.float32) * float(NROW) + pos_a,
                      float(TRASH))
    ia_ref[...] = idx_a.astype(jnp.int32)

    bc = bc_ref[...]
    eq_c = (bc == cols).astype(jnp.float32)
    cnt_c = jnp.sum(eq_c, axis=0, keepdims=True)
    starts_c = jax.lax.dot_general(cnt_c, lt, (((1,), (0,)), ((), ())),
                                   preferred_element_type=jnp.float32)
    base_c = jax.lax.dot_general(eq_c, (starts_c - cnt_a).reshape(B, 1),
                                 (((1,), (0,)), ((), ())),
                                 preferred_element_type=jnp.float32)
    pos_c = lax.broadcasted_iota(jnp.int32, (NCO, 1), 0).astype(jnp.float32) \
        - base_c
    idx_c = jnp.where(pos_c < float(NMAX),
                      bc.astype(jnp.float32) * float(NROW) + pos_c,
                      float(TRASH))
    ic_ref[...] = idx_c.astype(jnp.int32)

    t_ref[...] = (cnt_a + cnt_c).astype(jnp.int32)

  return pl.pallas_call(
      body,
      out_shape=[
          jax.ShapeDtypeStruct((NA, 1), jnp.int32),
          jax.ShapeDtypeStruct((NCO, 1), jnp.int32),
          jax.ShapeDtypeStruct((1, B), jnp.int32),
      ],
  )(batch_atoms.reshape(NA, 1), batch_cohps.reshape(NCO, 1))


def _mha_block(q_in, kv, Wq, Wk, Wv, Wo, klen, nheads=4):
  dh = D // nheads
  q = jax.lax.dot_general(q_in, Wq, (((1,), (0,)), ((), ())),
                          preferred_element_type=jnp.float32)
  k = jax.lax.dot_general(kv, Wk, (((1,), (0,)), ((), ())),
                          preferred_element_type=jnp.float32)
  v = jax.lax.dot_general(kv, Wv, (((1,), (0,)), ((), ())),
                          preferred_element_type=jnp.float32)
  outs = []
  for h in range(nheads):
    qh = q[:, h * dh:(h + 1) * dh]
    kh = k[:, h * dh:(h + 1) * dh]
    vh = v[:, h * dh:(h + 1) * dh]
    lg = jax.lax.dot_general(qh, kh, (((1,), (1,)), ((), ())),
                             preferred_element_type=jnp.float32)
    lg = lg * (1.0 / (dh ** 0.5))
    if klen is not None:
      colmask = lax.broadcasted_iota(jnp.int32, lg.shape, 1) < klen
      lg = jnp.where(colmask, lg, jnp.float32(-1e9))
    mx = jnp.max(lg, axis=1, keepdims=True)
    ex = jnp.exp(lg - mx)
    a = ex / jnp.sum(ex, axis=1, keepdims=True)
    outs.append(jax.lax.dot_general(a, vh, (((1,), (0,)), ((), ())),
                                    preferred_element_type=jnp.float32))
  o = jnp.concatenate(outs, axis=1)
  return jax.lax.dot_general(o, Wo, (((1,), (0,)), ((), ())),
                             preferred_element_type=jnp.float32)


def _tc_gmt(dense, tcnt, S1, S2, Ws, W3, b3):
  def body(t_ref, x_ref, s1_ref, s2_ref,
           q1w, k1w, v1w, o1w, q2w, k2w, v2w, o2w, q3w, k3w, v3w, o3w,
           w3_ref, b3_ref, out_ref):
    b = pl.program_id(0)
    m = jnp.minimum(t_ref[0, b], NMAX)
    rows = lax.broadcasted_iota(jnp.int32, (NROW, 1), 0)
    x = jnp.where(rows < m, x_ref[...], 0.0)
    h1 = jnp.maximum(
        _mha_block(s1_ref[...], x, q1w[...], k1w[...], v1w[...], o1w[...], m),
        0.0)
    h2 = jnp.maximum(
        _mha_block(h1, h1, q2w[...], k2w[...], v2w[...], o2w[...], None), 0.0)
    h3 = _mha_block(s2_ref[...], h2, q3w[...], k3w[...], v3w[...], o3w[...],
                    None)
    y = jax.lax.dot_general(h3, w3_ref[...], (((1,), (0,)), ((), ())),
                            preferred_element_type=jnp.float32) + b3_ref[...]
    out_ref[...] = jnp.broadcast_to(y.reshape(1, 1, 1), (1, 8, 128))

  wspec = pl.BlockSpec((D, D), lambda b: (0, 0))
  return pl.pallas_call(
      body,
      grid=(B,),
      in_specs=[
          pl.BlockSpec(memory_space=pltpu.SMEM),
          pl.BlockSpec((NROW, D), lambda b: (b, 0)),
          pl.BlockSpec((K1, D), lambda b: (0, 0)),
          pl.BlockSpec((1, D), lambda b: (0, 0)),
      ] + [wspec] * 12 + [
          pl.BlockSpec((D, 1), lambda b: (0, 0)),
          pl.BlockSpec((1, 1), lambda b: (0, 0)),
      ],
      out_specs=pl.BlockSpec((1, 8, 128), lambda b: (b, 0, 0)),
      out_shape=jax.ShapeDtypeStruct((B, 8, 128), jnp.float32),
  )(tcnt, dense, S1, S2, *Ws, W3, b3.reshape(1, 1))[:, 0, :1]


def kernel(x_atoms, x_cohps, src_a2c, dst_a2c, src_c2a, dst_c2a,
           batch_atoms, batch_cohps, W_node, b_node, W_edge, b_edge,
           Wsa0, bsa0, Wse0, bse0, Wca0, Wac0,
           Wsa1, bsa1, Wse1, bse1, Wca1, Wac1,
           Wsa2, bsa2, Wse2, bse2, Wca2, Wac2,
           S1, S2,
           Wq1, Wk1, Wv1, Wo1,
           Wq2, Wk2, Wv2, Wo2,
           Wq3, Wk3, Wv3, Wo3,
           W3, b3):
  ones_chunk = jnp.ones((EDGE_CHUNK, 16), jnp.float32)
  zeros_a16 = jnp.zeros((NA_PAD, 16), jnp.float32)
  zeros_c16 = jnp.zeros((NCO_PAD, 16), jnp.float32)

  def padw(W, po):
    pi = 128 if W.shape[0] in (96,) else W.shape[0]
    out = jnp.zeros((pi, po), jnp.float32)
    return out.at[:W.shape[0], :W.shape[1]].set(W)

  def padb(b, po):
    out = jnp.zeros((po,), jnp.float32)
    return out.at[:b.shape[0]].set(b)

  Wsa0, bsa0, Wse0, bse0 = padw(Wsa0, 128), padb(bsa0, 128), padw(Wse0, 128), padb(bse0, 128)
  Wca0, Wac0 = padw(Wca0, 128), padw(Wac0, 128)
  Wsa1, Wse1, Wca1, Wac1 = padw(Wsa1, 128), padw(Wse1, 128), padw(Wca1, 128), padw(Wac1, 128)

  xa = _tc_proj(x_atoms, padw(W_node, 128), padb(b_node, 128))
  xc = _tc_proj(x_cohps, padw(W_edge, 128), padb(b_edge, 128))

  degp = _sc_degrees(dst_c2a, dst_a2c, ones_chunk, zeros_a16, zeros_c16)
  deg_a = degp[:, :NA]
  deg_c = degp[:, NA_PAD:NA_PAD + NCO]

  layer_w = [
      (Wsa0, bsa0, Wse0, bse0, Wca0, Wac0),
      (Wsa1, bsa1, Wse1, bse1, Wca1, Wac1),
      (Wsa2, bsa2, Wse2, bse2, Wca2, Wac2),
  ]
  epad = E_PAD - E
  def pad_idx(idx, fill):
    return jnp.concatenate(
        [idx, jnp.full((epad,), fill, jnp.int32)]).reshape(-1, EDGE_CHUNK)

  sA2, dA2 = pad_idx(src_c2a, 0), pad_idx(dst_c2a, NA)
  sC2, dC2 = pad_idx(src_a2c, 0), pad_idx(dst_a2c, NCO)
  for l in range(3):
    Wsa, bsa, Wse, bse, Wca, Wac = layer_w[l]
    P_a = _sc_segsum(xc, sA2, dA2, NA_PAD, grp=1, idxch=40)
    P_c = _sc_segsum(xa, sC2, dC2, NCO_PAD, grp=1, idxch=80)
    xa_new = _tc_node_update(xa, P_a, deg_a, Wsa, bsa, Wca)
    xc_new = _tc_node_update(xc, P_c, deg_c, Wse, bse, Wac)
    xa, xc = xa_new, xc_new

  idx_a, idx_c, tcnt = _tc_pack_index(batch_atoms, batch_cohps)
  dense = _sc_pack(xa, xc, idx_a.reshape(NA), idx_c.reshape(NCO))

  Ws = (Wq1, Wk1, Wv1, Wo1, Wq2, Wk2, Wv2, Wo2, Wq3, Wk3, Wv3, Wo3)
  return _tc_gmt(dense, tcnt, S1, S2, Ws, W3, b3)


# speedup vs baseline: 2.0049x; 2.0049x over previous
import functools

import jax
import jax.numpy as jnp
from jax import lax
from jax.experimental import pallas as pl
from jax.experimental.pallas import tpu as pltpu
from jax.experimental.pallas import tpu_sc as plsc

NA = 10000
NCO = 5000
E = 320000
B = 64
NMAX = 450
K1 = 75
D = 256
DIMS = [96, 96, 128, 256]

NROW = 512
TRASH = B * NROW
DENSE_ROWS = B * NROW + 512
NA_PAD = 10240
NCO_PAD = 5120

SC_CORES = 2
SC_SUBCORES = 16
SC_WORKERS = SC_CORES * SC_SUBCORES
EDGE_CHUNK = 128
E_PAD = 327680


def _sc_mesh():
  return plsc.VectorSubcoreMesh(core_axis_name="c", subcore_axis_name="s")


def _sc_segsum(table, src, dst, n_pad, zeros, dep):
  nchunk = E // EDGE_CHUNK
  per_w = -(-nchunk // SC_WORKERS)
  rows_sub = n_pad // SC_SUBCORES

  @functools.partial(
      pl.kernel,
      out_type=jax.ShapeDtypeStruct((SC_CORES, n_pad, 128), jnp.float32),
      mesh=_sc_mesh(),
      scratch_types=[
          pltpu.VMEM((EDGE_CHUNK,), jnp.int32),
          pltpu.VMEM((EDGE_CHUNK,), jnp.int32),
          pltpu.VMEM((EDGE_CHUNK, 128), jnp.float32),
          pltpu.VMEM_SHARED((n_pad, 128), jnp.float32),
          pltpu.SemaphoreType.DMA,
      ],
  )
  def k(table_h, src_h, dst_h, zeros_h, dep_h, out_h,
        src_v, dst_v, rows_v, acc, sem):
    del dep_h
    c = lax.axis_index("c")
    s = lax.axis_index("s")
    wid = c * SC_SUBCORES + s
    pltpu.sync_copy(zeros_h.at[pl.ds(s * rows_sub, rows_sub)],
                    acc.at[pl.ds(s * rows_sub, rows_sub)])
    plsc.subcore_barrier()

    def body(g, carry):
      ch = wid + SC_WORKERS * g

      @pl.when(ch < nchunk)
      def _():
        off = ch * EDGE_CHUNK
        pltpu.sync_copy(src_h.at[pl.ds(off, EDGE_CHUNK)], src_v)
        pltpu.sync_copy(dst_h.at[pl.ds(off, EDGE_CHUNK)], dst_v)
        pltpu.async_copy(table_h.at[src_v], rows_v, sem).wait()
        pltpu.sync_copy(rows_v, acc.at[dst_v], add=True)

      return carry

    lax.fori_loop(0, per_w, body, 0)
    plsc.subcore_barrier()
    pltpu.sync_copy(acc.at[pl.ds(s * rows_sub, rows_sub)],
                    out_h.at[c, pl.ds(s * rows_sub, rows_sub)])

  return k(table, src, dst, zeros, dep)


def _sc_pack(xa, xc, idx_a, idx_c):
  CH = 128
  a_full = NA // CH
  a_tail = NA - a_full * CH
  c_full = NCO // CH
  c_tail = NCO - c_full * CH
  a_per_w = -(-a_full // SC_WORKERS)
  c_per_w = -(-c_full // SC_WORKERS)

  @functools.partial(
      pl.kernel,
      out_type=jax.ShapeDtypeStruct((DENSE_ROWS, D), jnp.float32),
      mesh=_sc_mesh(),
      scratch_types=[
          pltpu.VMEM((CH,), jnp.int32),
          pltpu.VMEM((CH, D), jnp.float32),
          pltpu.VMEM((a_tail,), jnp.int32),
          pltpu.VMEM((a_tail, D), jnp.float32),
          pltpu.VMEM((c_tail,), jnp.int32),
          pltpu.VMEM((c_tail, D), jnp.float32),
      ],
  )
  def k(xa_h, xc_h, ia_h, ic_h, out_h,
        idx_v, rows_v, iat_v, rat_v, ict_v, rct_v):
    c = lax.axis_index("c")
    s = lax.axis_index("s")
    wid = c * SC_SUBCORES + s

    def body_a(j, carry):
      ch = wid + SC_WORKERS * j

      @pl.when(ch < a_full)
      def _():
        off = ch * CH
        pltpu.sync_copy(ia_h.at[pl.ds(off, CH)], idx_v)
        pltpu.sync_copy(xa_h.at[pl.ds(off, CH)], rows_v)
        pltpu.sync_copy(rows_v, out_h.at[idx_v])

      return carry

    def body_c(j, carry):
      ch = wid + SC_WORKERS * j

      @pl.when(ch < c_full)
      def _():
        off = ch * CH
        pltpu.sync_copy(ic_h.at[pl.ds(off, CH)], idx_v)
        pltpu.sync_copy(xc_h.at[pl.ds(off, CH)], rows_v)
        pltpu.sync_copy(rows_v, out_h.at[idx_v])

      return carry

    lax.fori_loop(0, a_per_w, body_a, 0)
    lax.fori_loop(0, c_per_w, body_c, 0)

    @pl.when(wid == SC_WORKERS - 1)
    def _():
      pltpu.sync_copy(ia_h.at[pl.ds(a_full * CH, a_tail)], iat_v)
      pltpu.sync_copy(xa_h.at[pl.ds(a_full * CH, a_tail)], rat_v)
      pltpu.sync_copy(rat_v, out_h.at[iat_v])

    @pl.when(wid == SC_WORKERS - 2)
    def _():
      pltpu.sync_copy(ic_h.at[pl.ds(c_full * CH, c_tail)], ict_v)
      pltpu.sync_copy(xc_h.at[pl.ds(c_full * CH, c_tail)], rct_v)
      pltpu.sync_copy(rct_v, out_h.at[ict_v])

  return k(xa, xc, idx_a, idx_c)


def _tc_proj(x, W, b, ones_col=None):
  n, di = x.shape
  do = W.shape[1]
  BR = 1000
  grid = n // BR

  def body(x_ref, w_ref, b_ref, o_ref):
    y = jax.lax.dot_general(x_ref[...], w_ref[...], (((1,), (0,)), ((), ())),
                            preferred_element_type=jnp.float32)
    y = jnp.maximum(y + b_ref[...], 0.0)
    if ones_col is not None:
      cols = lax.broadcasted_iota(jnp.int32, y.shape, 1)
      y = jnp.where(cols == ones_col, 1.0, y)
    o_ref[...] = y

  return pl.pallas_call(
      body,
      grid=(grid,),
      in_specs=[
          pl.BlockSpec((BR, di), lambda i: (i, 0)),
          pl.BlockSpec((di, do), lambda i: (0, 0)),
          pl.BlockSpec((1, do), lambda i: (0, 0)),
      ],
      out_specs=pl.BlockSpec((BR, do), lambda i: (i, 0)),
      out_shape=jax.ShapeDtypeStruct((n, do), jnp.float32),
  )(x, W, b.reshape(1, do))


def _tc_node_update(x, P, degp, W_self, b_self, W_msg):
  n, di = x.shape
  do = W_self.shape[1]
  BR = 1000
  grid = n // BR

  def body(x_ref, p_ref, d_ref, ws_ref, b_ref, wm_ref, o_ref):
    deg = jnp.maximum(d_ref[0, :, 0:1] + d_ref[1, :, 0:1], 1.0)
    msg = (p_ref[0] + p_ref[1]) / deg
    y = jax.lax.dot_general(x_ref[...], ws_ref[...], (((1,), (0,)), ((), ())),
                            preferred_element_type=jnp.float32)
    y = y + jax.lax.dot_general(msg, wm_ref[...], (((1,), (0,)), ((), ())),
                                preferred_element_type=jnp.float32)
    o_ref[...] = jnp.maximum(y + b_ref[...], 0.0)

  return pl.pallas_call(
      body,
      grid=(grid,),
      in_specs=[
          pl.BlockSpec((BR, di), lambda i: (i, 0)),
          pl.BlockSpec((2, BR, di), lambda i: (0, i, 0)),
          pl.BlockSpec((2, BR, 16), lambda i: (0, i, 0)),
          pl.BlockSpec((di, do), lambda i: (0, 0)),
          pl.BlockSpec((1, do), lambda i: (0, 0)),
          pl.BlockSpec((di, do), lambda i: (0, 0)),
      ],
      out_specs=pl.BlockSpec((BR, do), lambda i: (i, 0)),
      out_shape=jax.ShapeDtypeStruct((n, do), jnp.float32),
  )(x, P, degp, W_self, b_self.reshape(1, do), W_msg)


def _tc_pack_index(batch_atoms, batch_cohps):
  def body(ba_ref, bc_ref, ia_ref, ic_ref, t_ref):
    cols = lax.broadcasted_iota(jnp.int32, (1, B), 1)
    ltmask = (lax.broadcasted_iota(jnp.int32, (B, B), 0)
              < lax.broadcasted_iota(jnp.int32, (B, B), 1))

    def starts_of(cnt):
      return jnp.sum(jnp.where(ltmask, jnp.broadcast_to(cnt.reshape(B, 1),
                                                        (B, B)), 0),
                     axis=0, keepdims=True)

    def pick(eq, vec):
      return jnp.sum(jnp.where(eq, vec, 0), axis=1, keepdims=True)

    ba = ba_ref[...]
    eq_a = ba == cols
    cnt_a = jnp.sum(eq_a.astype(jnp.int32), axis=0, keepdims=True)
    starts_a = starts_of(cnt_a)
    pos_a = (lax.broadcasted_iota(jnp.int32, (NA, 1), 0)
             - pick(eq_a, starts_a))
    ia_ref[...] = jnp.where(pos_a < NMAX, ba * NROW + pos_a, TRASH)

    bc = bc_ref[...]
    eq_c = bc == cols
    cnt_c = jnp.sum(eq_c.astype(jnp.int32), axis=0, keepdims=True)
    starts_c = starts_of(cnt_c)
    pos_c = (lax.broadcasted_iota(jnp.int32, (NCO, 1), 0)
             - pick(eq_c, starts_c - cnt_a))
    ic_ref[...] = jnp.where(pos_c < NMAX, bc * NROW + pos_c, TRASH)

    t_ref[...] = cnt_a + cnt_c

  return pl.pallas_call(
      body,
      out_shape=[
          jax.ShapeDtypeStruct((NA, 1), jnp.int32),
          jax.ShapeDtypeStruct((NCO, 1), jnp.int32),
          jax.ShapeDtypeStruct((1, B), jnp.int32),
      ],
  )(batch_atoms.reshape(NA, 1), batch_cohps.reshape(NCO, 1))


def _mha_block(q_in, kv, Wq, Wk, Wv, Wo, klen, nheads=4):
  dh = D // nheads
  q = jax.lax.dot_general(q_in, Wq, (((1,), (0,)), ((), ())),
                          preferred_element_type=jnp.float32)
  k = jax.lax.dot_general(kv, Wk, (((1,), (0,)), ((), ())),
                          preferred_element_type=jnp.float32)
  v = jax.lax.dot_general(kv, Wv, (((1,), (0,)), ((), ())),
                          preferred_element_type=jnp.float32)
  outs = []
  for h in range(nheads):
    qh = q[:, h * dh:(h + 1) * dh]
    kh = k[:, h * dh:(h + 1) * dh]
    vh = v[:, h * dh:(h + 1) * dh]
    lg = jax.lax.dot_general(qh, kh, (((1,), (1,)), ((), ())),
                             preferred_element_type=jnp.float32)
    lg = lg * (1.0 / (dh ** 0.5))
    if klen is not None:
      colmask = lax.broadcasted_iota(jnp.int32, lg.shape, 1) < klen
      lg = jnp.where(colmask, lg, jnp.float32(-1e9))
    mx = jnp.max(lg, axis=1, keepdims=True)
    ex = jnp.exp(lg - mx)
    a = ex / jnp.sum(ex, axis=1, keepdims=True)
    outs.append(jax.lax.dot_general(a, vh, (((1,), (0,)), ((), ())),
                                    preferred_element_type=jnp.float32))
  o = jnp.concatenate(outs, axis=1)
  return jax.lax.dot_general(o, Wo, (((1,), (0,)), ((), ())),
                             preferred_element_type=jnp.float32)


def _tc_gmt(dense, tcnt, S1, S2, Ws, W3, b3):
  def body(t_ref, x_ref, s1_ref, s2_ref,
           q1w, k1w, v1w, o1w, q2w, k2w, v2w, o2w, q3w, k3w, v3w, o3w,
           w3_ref, b3_ref, out_ref):
    b = pl.program_id(0)
    m = jnp.minimum(t_ref[0, b], NMAX)
    rows = lax.broadcasted_iota(jnp.int32, (NROW, 1), 0)
    x = jnp.where(rows < m, x_ref[...], 0.0)
    h1 = jnp.maximum(
        _mha_block(s1_ref[...], x, q1w[...], k1w[...], v1w[...], o1w[...], m),
        0.0)
    h2 = jnp.maximum(
        _mha_block(h1, h1, q2w[...], k2w[...], v2w[...], o2w[...], None), 0.0)
    h3 = _mha_block(s2_ref[...], h2, q3w[...], k3w[...], v3w[...], o3w[...],
                    None)
    y = jax.lax.dot_general(h3, w3_ref[...], (((1,), (0,)), ((), ())),
                            preferred_element_type=jnp.float32) + b3_ref[...]
    out_ref[...] = jnp.broadcast_to(y.reshape(1, 1, 1), (1, 8, 128))

  wspec = pl.BlockSpec((D, D), lambda b: (0, 0))
  return pl.pallas_call(
      body,
      grid=(B,),
      in_specs=[
          pl.BlockSpec(memory_space=pltpu.SMEM),
          pl.BlockSpec((NROW, D), lambda b: (b, 0)),
          pl.BlockSpec((K1, D), lambda b: (0, 0)),
          pl.BlockSpec((1, D), lambda b: (0, 0)),
      ] + [wspec] * 12 + [
          pl.BlockSpec((D, 1), lambda b: (0, 0)),
          pl.BlockSpec((1, 1), lambda b: (0, 0)),
      ],
      out_specs=pl.BlockSpec((1, 8, 128), lambda b: (b, 0, 0)),
      out_shape=jax.ShapeDtypeStruct((B, 8, 128), jnp.float32),
  )(tcnt, dense, S1, S2, *Ws, W3, b3.reshape(1, 1))[:, 0, :1]


def kernel(x_atoms, x_cohps, src_a2c, dst_a2c, src_c2a, dst_c2a,
           batch_atoms, batch_cohps, W_node, b_node, W_edge, b_edge,
           Wsa0, bsa0, Wse0, bse0, Wca0, Wac0,
           Wsa1, bsa1, Wse1, bse1, Wca1, Wac1,
           Wsa2, bsa2, Wse2, bse2, Wca2, Wac2,
           S1, S2,
           Wq1, Wk1, Wv1, Wo1,
           Wq2, Wk2, Wv2, Wo2,
           Wq3, Wk3, Wv3, Wo3,
           W3, b3):
  def padw(W, po):
    pi = 128 if W.shape[0] in (96,) else W.shape[0]
    out = jnp.zeros((pi, po), jnp.float32)
    return out.at[:W.shape[0], :W.shape[1]].set(W)

  def padb(b, po):
    out = jnp.zeros((po,), jnp.float32)
    return out.at[:b.shape[0]].set(b)

  Wsa0, bsa0, Wse0, bse0 = padw(Wsa0, 128), padb(bsa0, 128), padw(Wse0, 128), padb(bse0, 128)
  Wca0, Wac0 = padw(Wca0, 128), padw(Wac0, 128)
  Wsa1, Wse1, Wca1, Wac1 = padw(Wsa1, 128), padw(Wse1, 128), padw(Wca1, 128), padw(Wac1, 128)

  xa = _tc_proj(x_atoms, padw(W_node, 128), padb(b_node, 128), ones_col=96)
  xc = _tc_proj(x_cohps, padw(W_edge, 128), padb(b_edge, 128), ones_col=96)

  layer_w = [
      (Wsa0, bsa0, Wse0, bse0, Wca0, Wac0),
      (Wsa1, bsa1, Wse1, bse1, Wca1, Wac1),
      (Wsa2, bsa2, Wse2, bse2, Wca2, Wac2),
  ]
  zeros_ad = jnp.zeros((NA_PAD, 128), jnp.float32)
  zeros_cd = jnp.zeros((NCO_PAD, 128), jnp.float32)
  deg_a = deg_c = None
  dep = x_atoms
  for l in range(3):
    Wsa, bsa, Wse, bse, Wca, Wac = layer_w[l]
    P_a = _sc_segsum(xc, src_c2a, dst_c2a, NA_PAD, zeros_ad, dep)
    P_c = _sc_segsum(xa, src_a2c, dst_a2c, NCO_PAD, zeros_cd, P_a)
    dep = P_c
    if l == 0:
      deg_a = P_a[:, :NA, 96:112]
      deg_c = P_c[:, :NCO, 96:112]
    xa_new = _tc_node_update(xa, P_a, deg_a, Wsa, bsa, Wca)
    xc_new = _tc_node_update(xc, P_c, deg_c, Wse, bse, Wac)
    xa, xc = xa_new, xc_new

  idx_a, idx_c, tcnt = _tc_pack_index(batch_atoms, batch_cohps)
  dense = _sc_pack(xa, xc, idx_a.reshape(NA), idx_c.reshape(NCO))

  Ws = (Wq1, Wk1, Wv1, Wo1, Wq2, Wk2, Wv2, Wo2, Wq3, Wk3, Wv3, Wo3)
  return _tc_gmt(dense, tcnt, S1, S2, Ws, W3, b3)
